# baseline (device time: 22375 ns/iter reference)
import jax
import jax.numpy as jnp
from jax import lax
from jax.experimental import pallas as pl
from jax.experimental.pallas import tpu as pltpu

CHUNK_ROWS = (96, 96, 64, 64, 64, 64, 32, 32)
C = len(CHUNK_ROWS)
OFFS = tuple(sum(CHUNK_ROWS[:i]) for i in range(C))
S = 5.0 / 127.0
SI = 127.0 / 5.0

_HBM = pl.BlockSpec(memory_space=pltpu.MemorySpace.HBM)


def kernel(partial, resid, gamma):
    _, m, d = partial.shape
    h = m // 2

    def body(partial_ref, resid_ref, gamma_ref, out_ref,
             part_v, res_v, gam_v,
             rsq_send, rsq_recv, agq_send, agq_recv,
             my_out, xp_out,
             part_sem, res_sem, gam_sem,
             rsq_ssem, rsq_rsem, agq_ssem, agq_rsem,
             myout_sem, xpout_sem):
        my_x = lax.axis_index("x")
        my_y = lax.axis_index("y")
        ypeer = (my_x, 1 - my_y)
        xpeer = (1 - my_x, my_y)
        my_half = h * my_x
        other_half = h * (1 - my_x)

        gam_cp = pltpu.make_async_copy(gamma_ref, gam_v, gam_sem)
        gam_cp.start()
        part_cps, res_cps = [], []
        for c in range(C):
            sl = slice(OFFS[c], OFFS[c] + CHUNK_ROWS[c])
            rows = pl.ds(my_half + OFFS[c], CHUNK_ROWS[c])
            p = pltpu.make_async_copy(
                partial_ref.at[0, rows, :], part_v.at[sl], part_sem.at[c])
            p.start()
            part_cps.append(p)
            r = pltpu.make_async_copy(
                resid_ref.at[rows, :], res_v.at[sl], res_sem.at[c])
            r.start()
            res_cps.append(r)

        barrier = pltpu.get_barrier_semaphore()
        for nbr in (ypeer, xpeer):
            pl.semaphore_signal(barrier, inc=1, device_id=nbr,
                                device_id_type=pl.DeviceIdType.MESH)
        pl.semaphore_wait(barrier, 2)

        rsq = []
        for c in range(C):
            sl = slice(OFFS[c], OFFS[c] + CHUNK_ROWS[c])
            part_cps[c].wait()
            rsq_send[sl, :] = jnp.clip(
                jnp.round(part_v[sl, :] * SI), -127.0, 127.0
            ).astype(jnp.int8)
            r = pltpu.make_async_remote_copy(
                src_ref=rsq_send.at[sl], dst_ref=rsq_recv.at[sl],
                send_sem=rsq_ssem.at[c], recv_sem=rsq_rsem.at[c],
                device_id=ypeer, device_id_type=pl.DeviceIdType.MESH)
            r.start()
            rsq.append(r)

        gam_cp.wait()
        agq, myout_cps = [], []
        for c in range(C):
            sl = slice(OFFS[c], OFFS[c] + CHUNK_ROWS[c])
            res_cps[c].wait()
            rsq[c].wait_recv()
            yv = (part_v[sl, :] + rsq_recv[sl, :].astype(jnp.float32) * S
                  + res_v[sl, :])
            rinv = lax.rsqrt(jnp.mean(yv * yv, axis=-1, keepdims=True)
                             + 1e-6)
            yhat = yv * rinv
            agq_send[sl, :] = jnp.clip(
                jnp.round(yhat * SI), -127.0, 127.0
            ).astype(jnp.int8)
            r = pltpu.make_async_remote_copy(
                src_ref=agq_send.at[sl], dst_ref=agq_recv.at[sl],
                send_sem=agq_ssem.at[c], recv_sem=agq_rsem.at[c],
                device_id=xpeer, device_id_type=pl.DeviceIdType.MESH)
            r.start()
            agq.append(r)
            my_out[sl, :] = yhat * gam_v[...]
            o = pltpu.make_async_copy(
                my_out.at[sl],
                out_ref.at[pl.ds(my_half + OFFS[c], CHUNK_ROWS[c]), :],
                myout_sem.at[c])
            o.start()
            myout_cps.append(o)

        xpout_cps = []
        for c in range(C):
            sl = slice(OFFS[c], OFFS[c] + CHUNK_ROWS[c])
            agq[c].wait_recv()
            xp_out[sl, :] = (agq_recv[sl, :].astype(jnp.float32) * S
                             * gam_v[...])
            o = pltpu.make_async_copy(
                xp_out.at[sl],
                out_ref.at[pl.ds(other_half + OFFS[c], CHUNK_ROWS[c]), :],
                xpout_sem.at[c])
            o.start()
            xpout_cps.append(o)

        for c in range(C):
            rsq[c].wait_send()
            agq[c].wait_send()
            myout_cps[c].wait()
            xpout_cps[c].wait()

    return pl.pallas_call(
        body,
        out_shape=jax.ShapeDtypeStruct((m, d), jnp.float32),
        in_specs=[_HBM, _HBM, _HBM],
        out_specs=_HBM,
        scratch_shapes=[
            pltpu.VMEM((h, d), jnp.float32),
            pltpu.VMEM((h, d), jnp.float32),
            pltpu.VMEM((1, d), jnp.float32),
            pltpu.VMEM((h, d), jnp.int8),
            pltpu.VMEM((h, d), jnp.int8),
            pltpu.VMEM((h, d), jnp.int8),
            pltpu.VMEM((h, d), jnp.int8),
            pltpu.VMEM((h, d), jnp.float32),
            pltpu.VMEM((h, d), jnp.float32),
            pltpu.SemaphoreType.DMA((C,)),
            pltpu.SemaphoreType.DMA((C,)),
            pltpu.SemaphoreType.DMA,
            pltpu.SemaphoreType.DMA((C,)),
            pltpu.SemaphoreType.DMA((C,)),
            pltpu.SemaphoreType.DMA((C,)),
            pltpu.SemaphoreType.DMA((C,)),
            pltpu.SemaphoreType.DMA((C,)),
            pltpu.SemaphoreType.DMA((C,)),
        ],
        compiler_params=pltpu.CompilerParams(collective_id=0),
    )(partial, resid, gamma.reshape(1, d))


# device time: 20578 ns/iter; 1.0873x vs baseline; 1.0873x over previous
import jax
import jax.numpy as jnp
from jax import lax
from jax.experimental import pallas as pl
from jax.experimental.pallas import tpu as pltpu

C = 8
S = 5.0 / 127.0
SI = 127.0 / 5.0

_HBM = pl.BlockSpec(memory_space=pltpu.MemorySpace.HBM)


def kernel(partial, resid, gamma):
    _, m, d = partial.shape
    h = m // 2
    rc = h // C

    def body(partial_ref, resid_ref, gamma_ref, out_ref,
             part_v, res_v, gam_v,
             rsq_send, rsq_recv, agq_send, agq_recv,
             my_out, xp_out,
             part_sem, res_sem, gam_sem,
             rsq_ssem, rsq_rsem, agq_ssem, agq_rsem,
             myout_sem, xpout_sem):
        my_x = lax.axis_index("x")
        my_y = lax.axis_index("y")
        ypeer = (my_x, 1 - my_y)
        xpeer = (1 - my_x, my_y)
        my_half = h * my_x
        other_half = h * (1 - my_x)

        gam_cp = pltpu.make_async_copy(gamma_ref, gam_v, gam_sem)
        gam_cp.start()
        part_cps, res_cps = [], []
        for c in range(C):
            sl = slice(c * rc, (c + 1) * rc)
            rows = pl.ds(my_half + c * rc, rc)
            p = pltpu.make_async_copy(
                partial_ref.at[0, rows, :], part_v.at[sl], part_sem.at[c])
            p.start()
            part_cps.append(p)
            r = pltpu.make_async_copy(
                resid_ref.at[rows, :], res_v.at[sl], res_sem.at[c])
            r.start()
            res_cps.append(r)

        barrier = pltpu.get_barrier_semaphore()
        for nbr in (ypeer, xpeer):
            pl.semaphore_signal(barrier, inc=1, device_id=nbr,
                                device_id_type=pl.DeviceIdType.MESH)
        pl.semaphore_wait(barrier, 2)

        rsq = []
        for c in range(C):
            sl = slice(c * rc, (c + 1) * rc)
            part_cps[c].wait()
            rsq_send[sl, :] = jnp.clip(
                jnp.round(part_v[sl, :] * SI), -127.0, 127.0
            ).astype(jnp.int8)
            r = pltpu.make_async_remote_copy(
                src_ref=rsq_send.at[sl], dst_ref=rsq_recv.at[sl],
                send_sem=rsq_ssem.at[c], recv_sem=rsq_rsem.at[c],
                device_id=ypeer, device_id_type=pl.DeviceIdType.MESH)
            r.start()
            rsq.append(r)

        gam_cp.wait()
        agq, myout_cps = [], []
        for c in range(C):
            sl = slice(c * rc, (c + 1) * rc)
            res_cps[c].wait()
            rsq[c].wait_recv()
            yv = (part_v[sl, :] + rsq_recv[sl, :].astype(jnp.float32) * S
                  + res_v[sl, :])
            rinv = lax.rsqrt(jnp.mean(yv * yv, axis=-1, keepdims=True)
                             + 1e-6)
            yhat = yv * rinv
            agq_send[sl, :] = jnp.clip(
                jnp.round(yhat * SI), -127.0, 127.0
            ).astype(jnp.int8)
            r = pltpu.make_async_remote_copy(
                src_ref=agq_send.at[sl], dst_ref=agq_recv.at[sl],
                send_sem=agq_ssem.at[c], recv_sem=agq_rsem.at[c],
                device_id=xpeer, device_id_type=pl.DeviceIdType.MESH)
            r.start()
            agq.append(r)
            my_out[sl, :] = yhat * gam_v[...]
            o = pltpu.make_async_copy(
                my_out.at[sl], out_ref.at[pl.ds(my_half + c * rc, rc), :],
                myout_sem.at[c])
            o.start()
            myout_cps.append(o)

        xpout_cps = []
        for c in range(C):
            sl = slice(c * rc, (c + 1) * rc)
            agq[c].wait_recv()
            xp_out[sl, :] = (agq_recv[sl, :].astype(jnp.float32) * S
                             * gam_v[...])
            o = pltpu.make_async_copy(
                xp_out.at[sl], out_ref.at[pl.ds(other_half + c * rc, rc), :],
                xpout_sem.at[c])
            o.start()
            xpout_cps.append(o)

        for c in range(C):
            rsq[c].wait_send()
            agq[c].wait_send()
            myout_cps[c].wait()
            xpout_cps[c].wait()

    return pl.pallas_call(
        body,
        out_shape=jax.ShapeDtypeStruct((m, d), jnp.float32),
        in_specs=[_HBM, _HBM, _HBM],
        out_specs=_HBM,
        scratch_shapes=[
            pltpu.VMEM((h, d), jnp.float32),
            pltpu.VMEM((h, d), jnp.float32),
            pltpu.VMEM((1, d), jnp.float32),
            pltpu.VMEM((h, d), jnp.int8),
            pltpu.VMEM((h, d), jnp.int8),
            pltpu.VMEM((h, d), jnp.int8),
            pltpu.VMEM((h, d), jnp.int8),
            pltpu.VMEM((h, d), jnp.float32),
            pltpu.VMEM((h, d), jnp.float32),
            pltpu.SemaphoreType.DMA((C,)),
            pltpu.SemaphoreType.DMA((C,)),
            pltpu.SemaphoreType.DMA,
            pltpu.SemaphoreType.DMA((C,)),
            pltpu.SemaphoreType.DMA((C,)),
            pltpu.SemaphoreType.DMA((C,)),
            pltpu.SemaphoreType.DMA((C,)),
            pltpu.SemaphoreType.DMA((C,)),
            pltpu.SemaphoreType.DMA((C,)),
        ],
        compiler_params=pltpu.CompilerParams(collective_id=0),
    )(partial, resid, gamma.reshape(1, d))
